# baseline (speedup 1.0000x reference)
import jax
import jax.numpy as jnp
from jax import lax
from jax.experimental import pallas as pl
from jax.experimental.pallas import tpu as pltpu
from jax.experimental.pallas import tpu_sc as plsc

USER_N = 5000
ITEM_N = 5000
N = USER_N + ITEM_N
D = 128
DH = 64
NLAYERS = 3
E = 320000
NC = 2
NS = 16
CHUNK = 128
NCH = 160
EPT = NCH * CHUNK
E_PAD = NS * EPT
NP = 10240
RPT = NP // NS
RC = 128
ZR = 32
NB = 4
NG = NCH // NB
NEB = 4
NGB = 5
NSB = 3
GLA = 4


def _sc_body(emb_hbm, embh_hbm, edges_hbm, vals_hbm, out_hbm,
             ebuf, vbuf, gbufh, sbuf, abuf, zbuf, b_sh, cur_sh,
             esem, gsem, ssem):
    c_id = lax.axis_index("c")
    s_id = lax.axis_index("s")
    base = s_id * RPT
    cbase = c_id * NP + base

    def issue_e(g, slot):
        pltpu.async_copy(edges_hbm.at[s_id, g], ebuf.at[slot], esem.at[slot])
        pltpu.async_copy(vals_hbm.at[s_id, g], vbuf.at[slot], esem.at[slot])

    def wait_e(g, slot):
        pltpu.make_async_copy(edges_hbm.at[s_id, g], ebuf.at[slot],
                              esem.at[slot]).wait()
        pltpu.make_async_copy(vals_hbm.at[s_id, g], vbuf.at[slot],
                              esem.at[slot]).wait()

    def issue_g(slot, qq, gb):
        pltpu.async_copy(cur_sh.at[ebuf.at[slot, 0, qq]], gbufh.at[gb],
                         gsem.at[gb])

    def wait_g(gb):
        pltpu.make_async_copy(cur_sh.at[ebuf.at[0, 0, 0]], gbufh.at[gb],
                              gsem.at[gb]).wait()

    def issue_s(slot, qq, sb):
        pltpu.async_copy(sbuf.at[sb], b_sh.at[ebuf.at[slot, 1, qq]],
                         ssem.at[sb], add=True)

    def wait_s(sb):
        pltpu.make_async_copy(sbuf.at[0], b_sh.at[ebuf.at[0, 1, 0]],
                              ssem.at[sb]).wait()

    pltpu.sync_copy(emb_hbm.at[pl.ds(cbase, RPT)], out_hbm.at[pl.ds(cbase, RPT)])
    pltpu.sync_copy(embh_hbm.at[pl.ds(cbase, RPT)], cur_sh.at[pl.ds(base, RPT)])

    def _zb(i, carry):
        for q in range(4):
            zbuf[i, pl.ds(q * 16, 16)] = jnp.zeros((16,), jnp.float32)
        return carry
    lax.fori_loop(0, ZR, _zb, 0)
    for k in range(RPT // ZR):
        pltpu.sync_copy(zbuf, b_sh.at[pl.ds(base + k * ZR, ZR)])
    plsc.subcore_barrier()

    def _layer(l, carry):
        issue_e(0, 0)
        issue_e(1, 1)
        wait_e(0, 0)
        for j in range(NB):
            issue_g(0, j, j)

        def _group(g, carry2):
            j0 = g * NB
            slot1 = lax.rem(g + 1, NEB)
            @pl.when(g + 2 < NG)
            def _():
                issue_e(g + 2, lax.rem(g + 2, NEB))

            @pl.when(g + 1 < NG)
            def _():
                wait_e(g + 1, slot1)
            for b in range(NB):
                j = j0 + b
                slot = lax.rem(g, NEB)
                gb = lax.rem(j, NGB)
                sb = lax.rem(j, NSB)
                wait_g(gb)
                @pl.when(j + GLA < NCH)
                def _():
                    issue_g(slot1, b, lax.rem(j + GLA, NGB))
                @pl.when(j >= NSB)
                def _():
                    wait_s(sb)
                for grp in range(CHUNK // 16):
                    vv = vbuf[slot, b, pl.ds(grp * 16, 16)]
                    for i in range(16):
                        e = grp * 16 + i
                        v = jnp.take_along_axis(
                            vv, jnp.full((16,), i, jnp.int32), axis=0)
                        for h in range(2):
                            ab = gbufh[gb, e, pl.ds(h * 32, 32)]
                            x0, x1 = plsc.unpack(
                                ab, format=plsc.PackFormat.INTERLEAVED)
                            sbuf[sb, e, pl.ds(h * 32, 16)] = x0 * v
                            sbuf[sb, e, pl.ds(h * 32 + 16, 16)] = x1 * v
                issue_s(slot, b, sb)
            return carry2
        lax.fori_loop(0, NG, _group, 0)
        for j in range(NCH - NSB, NCH):
            wait_s(lax.rem(j, NSB))
        plsc.subcore_barrier()

        for k in range(RPT // RC):
            rb = base + k * RC
            cb = cbase + k * RC
            pltpu.sync_copy(b_sh.at[pl.ds(rb, RC)], sbuf.at[0])
            for z in range(RC // ZR):
                pltpu.sync_copy(zbuf, b_sh.at[pl.ds(rb + z * ZR, ZR)])
            pltpu.sync_copy(out_hbm.at[pl.ds(cb, RC)], abuf)

            def _acc(i, carry3):
                for h in range(2):
                    a0 = sbuf[0, i, pl.ds(h * 32, 16)]
                    a1 = sbuf[0, i, pl.ds(h * 32 + 16, 16)]
                    gbufh[0, i, pl.ds(h * 32, 32)] = plsc.pack(
                        a0, a1, format=plsc.PackFormat.INTERLEAVED)
                    sl0 = pl.ds(h * 32, 16)
                    sl1 = pl.ds(h * 32 + 16, 16)
                    abuf[i, sl0] = abuf[i, sl0] + a0
                    abuf[i, sl1] = abuf[i, sl1] + a1
                return carry3
            lax.fori_loop(0, RC, _acc, 0)
            pltpu.sync_copy(abuf, out_hbm.at[pl.ds(cb, RC)])
            pltpu.sync_copy(gbufh.at[0], cur_sh.at[pl.ds(rb, RC)])
        plsc.subcore_barrier()
        return carry

    lax.fori_loop(0, NLAYERS, _layer, 0)


def _make_call():
    mesh = plsc.VectorSubcoreMesh(core_axis_name="c", subcore_axis_name="s",
                                  num_cores=NC, num_subcores=NS)
    return pl.kernel(
        _sc_body,
        out_type=jax.ShapeDtypeStruct((NC * NP, DH), jnp.float32),
        mesh=mesh,
        compiler_params=pltpu.CompilerParams(use_tc_tiling_on_sc=False,
                                             needs_layout_passes=False),
        scratch_types=[
            pltpu.VMEM((NEB, 2, NB, CHUNK), jnp.int32),
            pltpu.VMEM((NEB, NB, CHUNK), jnp.float32),
            pltpu.VMEM((NGB, CHUNK, DH), jnp.bfloat16),
            pltpu.VMEM((NSB, CHUNK, DH), jnp.float32),
            pltpu.VMEM((RC, DH), jnp.float32),
            pltpu.VMEM((ZR, DH), jnp.float32),
            pltpu.VMEM_SHARED((NP, DH), jnp.float32),
            pltpu.VMEM_SHARED((NP, DH), jnp.bfloat16),
            pltpu.SemaphoreType.DMA((NEB,)),
            pltpu.SemaphoreType.DMA((NGB,)),
            pltpu.SemaphoreType.DMA((NSB,)),
        ],
    )


_sc_call = _make_call()


def _to_packed_bf16(x):
    r = x.shape[0]
    t = x.reshape(r, DH // 32, 2, 16)
    inter = jnp.stack([t[:, :, 0, :], t[:, :, 1, :]], axis=-1)
    return inter.reshape(r, DH).astype(jnp.bfloat16)


def kernel(adj_edge_index, adj_edge_values, uEmbeds, iEmbeds):
    embeds = jnp.concatenate([uEmbeds, iEmbeds], axis=0)
    rpad = jnp.zeros((NP - N, DH), jnp.float32)
    emb_flat = jnp.concatenate(
        [embeds[:, :DH], rpad, embeds[:, DH:], rpad], axis=0)
    emb_bh = _to_packed_bf16(emb_flat)

    dst = adj_edge_index[0]
    src = adj_edge_index[1]
    npad = E_PAD - E
    pad_idx = (jnp.arange(npad, dtype=jnp.int32) * 61) % N
    src_p = jnp.concatenate([src, pad_idx])
    dst_p = jnp.concatenate([dst, pad_idx])
    vals_p = jnp.concatenate([adj_edge_values,
                              jnp.zeros((npad,), jnp.float32)])

    edges = jnp.stack([src_p, dst_p])
    edges_a = edges.reshape(2, NS, NG, NB, CHUNK).transpose(1, 2, 0, 3, 4)
    vals_a = vals_p.reshape(NS, NG, NB, CHUNK)

    out_flat = _sc_call(emb_flat, emb_bh, edges_a, vals_a)
    out = jnp.concatenate([out_flat[:N], out_flat[NP:NP + N]], axis=1)
    return (out[:USER_N], out[USER_N:])


# speedup vs baseline: 2.4111x; 2.4111x over previous
import jax
import jax.numpy as jnp
from jax import lax
from jax.experimental import pallas as pl
from jax.experimental.pallas import tpu as pltpu
from jax.experimental.pallas import tpu_sc as plsc

USER_N = 5000
ITEM_N = 5000
N = USER_N + ITEM_N
D = 128
DH = 64
NLAYERS = 3
E = 320000
NC = 2
NS = 16
CHUNK = 128
NCH = 160
EPT = NCH * CHUNK
E_PAD = NS * EPT
NP = 10240
RPT = NP // NS
RC = 128
ZR = 32
NB = 4
NG = NCH // NB
NEB = 4


def _sc_body(emb_hbm, edges_hbm, vals_hbm, out_hbm,
             ebuf, vbuf, gbuf, abuf, zbuf, b_sh, cur_sh, esem, gsem, ssem):
    c_id = lax.axis_index("c")
    s_id = lax.axis_index("s")
    base = s_id * RPT
    cbase = c_id * NP + base

    def issue_e(g, slot):
        pltpu.async_copy(edges_hbm.at[s_id, g], ebuf.at[slot], esem.at[slot])
        pltpu.async_copy(vals_hbm.at[s_id, g], vbuf.at[slot], esem.at[slot])

    def wait_e(g, slot):
        pltpu.make_async_copy(edges_hbm.at[s_id, g], ebuf.at[slot],
                              esem.at[slot]).wait()
        pltpu.make_async_copy(vals_hbm.at[s_id, g], vbuf.at[slot],
                              esem.at[slot]).wait()

    def issue_g(slot, qq, b):
        pltpu.async_copy(cur_sh.at[ebuf.at[slot, 0, qq]], gbuf.at[b],
                         gsem.at[b])

    def wait_g(b):
        pltpu.make_async_copy(cur_sh.at[ebuf.at[0, 0, 0]], gbuf.at[b],
                              gsem.at[b]).wait()

    def issue_s(slot, qq, b):
        pltpu.async_copy(gbuf.at[b], b_sh.at[ebuf.at[slot, 1, qq]],
                         ssem.at[b], add=True)

    def wait_s(b):
        pltpu.make_async_copy(gbuf.at[b], b_sh.at[ebuf.at[0, 1, 0]],
                              ssem.at[b]).wait()

    pltpu.sync_copy(emb_hbm.at[pl.ds(cbase, RPT)], out_hbm.at[pl.ds(cbase, RPT)])
    pltpu.sync_copy(emb_hbm.at[pl.ds(cbase, RPT)], cur_sh.at[pl.ds(base, RPT)])

    def _zb(i, carry):
        for q in range(4):
            zbuf[i, pl.ds(q * 16, 16)] = jnp.zeros((16,), jnp.float32)
        return carry
    lax.fori_loop(0, ZR, _zb, 0)
    for k in range(RPT // ZR):
        pltpu.sync_copy(zbuf, b_sh.at[pl.ds(base + k * ZR, ZR)])
    plsc.subcore_barrier()

    def _layer(l, carry):
        issue_e(0, 0)
        issue_e(1, 1)
        wait_e(0, 0)
        issue_g(0, 0, 0)
        issue_g(0, 1, 1)

        def _group(g, carry2):
            j0 = g * NB
            slot = lax.rem(g, NEB)
            slot1 = lax.rem(g + 1, NEB)
            @pl.when(g + 2 < NG)
            def _():
                issue_e(g + 2, lax.rem(g + 2, NEB))
            for b in range(NB):
                j = j0 + b
                wait_g(b)
                for grp in range(CHUNK // 16):
                    vv = vbuf[slot, b, pl.ds(grp * 16, 16)]
                    for i in range(16):
                        e = grp * 16 + i
                        v = jnp.take_along_axis(
                            vv, jnp.full((16,), i, jnp.int32), axis=0)
                        for q in range(4):
                            sl = pl.ds(q * 16, 16)
                            gbuf[b, e, sl] = gbuf[b, e, sl] * v
                issue_s(slot, b, b)
                b2 = (b + 2) % NB
                @pl.when(j >= 2)
                def _():
                    wait_s(b2)
                if b == 2:
                    @pl.when(g + 1 < NG)
                    def _():
                        wait_e(g + 1, slot1)

                @pl.when(j + 2 < NCH)
                def _():
                    if b < 2:
                        issue_g(slot, b + 2, b2)
                    else:
                        issue_g(slot1, b - 2, b2)
            return carry2
        lax.fori_loop(0, NG, _group, 0)
        wait_s((NCH - 2) % NB)
        wait_s((NCH - 1) % NB)
        plsc.subcore_barrier()

        for k in range(RPT // RC):
            rb = base + k * RC
            cb = cbase + k * RC
            pltpu.sync_copy(b_sh.at[pl.ds(rb, RC)], gbuf.at[0])
            for z in range(RC // ZR):
                pltpu.sync_copy(zbuf, b_sh.at[pl.ds(rb + z * ZR, ZR)])
            pltpu.sync_copy(out_hbm.at[pl.ds(cb, RC)], abuf)

            def _acc(i, carry3):
                for q in range(4):
                    sl = pl.ds(q * 16, 16)
                    abuf[i, sl] = abuf[i, sl] + gbuf[0, i, sl]
                return carry3
            lax.fori_loop(0, RC, _acc, 0)
            pltpu.sync_copy(abuf, out_hbm.at[pl.ds(cb, RC)])
            pltpu.sync_copy(gbuf.at[0], cur_sh.at[pl.ds(rb, RC)])
        plsc.subcore_barrier()
        return carry

    lax.fori_loop(0, NLAYERS, _layer, 0)


def _make_call():
    mesh = plsc.VectorSubcoreMesh(core_axis_name="c", subcore_axis_name="s",
                                  num_cores=NC, num_subcores=NS)
    return pl.kernel(
        _sc_body,
        out_type=jax.ShapeDtypeStruct((NC * NP, DH), jnp.float32),
        mesh=mesh,
        compiler_params=pltpu.CompilerParams(use_tc_tiling_on_sc=False),
        scratch_types=[
            pltpu.VMEM((NEB, 2, NB, CHUNK), jnp.int32),
            pltpu.VMEM((NEB, NB, CHUNK), jnp.float32),
            pltpu.VMEM((NB, CHUNK, DH), jnp.float32),
            pltpu.VMEM((RC, DH), jnp.float32),
            pltpu.VMEM((ZR, DH), jnp.float32),
            pltpu.VMEM_SHARED((NP, DH), jnp.float32),
            pltpu.VMEM_SHARED((NP, DH), jnp.float32),
            pltpu.SemaphoreType.DMA((NEB,)),
            pltpu.SemaphoreType.DMA((NB,)),
            pltpu.SemaphoreType.DMA((NB,)),
        ],
    )


_sc_call = _make_call()


def kernel(adj_edge_index, adj_edge_values, uEmbeds, iEmbeds):
    embeds = jnp.concatenate([uEmbeds, iEmbeds], axis=0)
    rpad = jnp.zeros((NP - N, DH), jnp.float32)
    emb_flat = jnp.concatenate(
        [embeds[:, :DH], rpad, embeds[:, DH:], rpad], axis=0)

    dst = adj_edge_index[0]
    src = adj_edge_index[1]
    npad = E_PAD - E
    pad_idx = (jnp.arange(npad, dtype=jnp.int32) * 61) % N
    src_p = jnp.concatenate([src, pad_idx])
    dst_p = jnp.concatenate([dst, pad_idx])
    vals_p = jnp.concatenate([adj_edge_values,
                              jnp.zeros((npad,), jnp.float32)])

    edges = jnp.stack([src_p, dst_p])
    edges_a = edges.reshape(2, NS, NG, NB, CHUNK).transpose(1, 2, 0, 3, 4)
    vals_a = vals_p.reshape(NS, NG, NB, CHUNK)

    out_flat = _sc_call(emb_flat, edges_a, vals_a)
    out = jnp.concatenate([out_flat[:N], out_flat[NP:NP + N]], axis=1)
    return (out[:USER_N], out[USER_N:])
